# baseline (speedup 1.0000x reference)
import functools
import math

import jax
import jax.numpy as jnp
from jax import lax
from jax.experimental import pallas as pl
from jax.experimental.pallas import tpu as pltpu
from jax.experimental.pallas import tpu_sc as plsc

_NH = 8
_SB = 80
_NW = 32
_INV = 1.0 / math.sqrt(32.0)


def _splat(v):
    return jnp.full((16,), v, jnp.int32)


def _sc_pass1(k0T, k1T, qcat, ei2, zeros):
    E = k0T.shape[2]
    N = qcat.shape[0]
    C = 400
    NSB = C // _SB
    PW = E // _NW
    NCH = PW // C
    G = C // 16
    N16 = N // 16
    mesh = plsc.VectorSubcoreMesh(core_axis_name="c", subcore_axis_name="s")

    @functools.partial(
        pl.kernel,
        out_type=(jax.ShapeDtypeStruct((E, _NH), jnp.float32),
                  jax.ShapeDtypeStruct((2, N, _NH), jnp.float32)),
        mesh=mesh,
        compiler_params=pltpu.CompilerParams(use_tc_tiling_on_sc=False,
                                             needs_layout_passes=False),
        scratch_types=[
            pltpu.VMEM((1, _NH, C), jnp.float32),
            pltpu.VMEM((3, _NH, C), jnp.float32),
            pltpu.VMEM((C, 32), jnp.float32),
            pltpu.VMEM((C, _NH), jnp.float32),
            pltpu.VMEM((NSB, _SB), jnp.int32),
            pltpu.VMEM((NSB, _SB), jnp.int32),
            pltpu.VMEM_SHARED((N, _NH), jnp.float32),
        ],
    )
    def kern(k0_h, k1_h, q_h, ei_h, z_h, e_h, sp_h,
             k0b, k1b, qgb, ecb, idxr, idxc, s_sh):
        core = lax.axis_index("c")
        sub = lax.axis_index("s")
        wid = core * 16 + sub
        rw = wid * (PW // _SB)
        pltpu.sync_copy(z_h.at[pl.ds(sub * N16, N16), pl.ds(0, _NH)],
                        s_sh.at[pl.ds(sub * N16, N16)])
        plsc.subcore_barrier()
        iota = lax.iota(jnp.int32, 16)

        def compute():
            @pl.loop(0, G)
            def _group(g):
                cv = iota + g * 16
                for h in range(_NH):
                    acc = (k0b[0, h, pl.ds(g * 16, 16)] *
                           plsc.load_gather(qgb, [cv, _splat(4 * h)]))
                    for d in range(3):
                        acc = acc + (
                            k1b[d, h, pl.ds(g * 16, 16)] *
                            plsc.load_gather(qgb,
                                             [cv, _splat(4 * h + 1 + d)]))
                    plsc.store_scatter(ecb, [cv, _splat(h)],
                                       jnp.exp(acc * _INV))

        @pl.loop(0, NCH)
        def _chunk(i):
            eb = wid * PW + i * C
            rb = rw + i * NSB
            pltpu.sync_copy(k0_h.at[:, :, pl.ds(eb, C)], k0b)
            pltpu.sync_copy(k1_h.at[:, :, pl.ds(eb, C)], k1b)
            pltpu.sync_copy(ei_h.at[pl.ds(rb, NSB)], idxr)
            pltpu.sync_copy(ei_h.at[pl.ds(E // _SB + rb, NSB)], idxc)
            for j in range(NSB):
                pltpu.sync_copy(q_h.at[idxc.at[j]],
                                qgb.at[pl.ds(j * _SB, _SB)])
            compute()
            pltpu.sync_copy(ecb, e_h.at[pl.ds(eb, C)])
            for j in range(NSB):
                pltpu.sync_copy(ecb.at[pl.ds(j * _SB, _SB)],
                                s_sh.at[idxr.at[j]], add=True)

        plsc.subcore_barrier()
        pltpu.sync_copy(s_sh.at[pl.ds(sub * N16, N16)],
                        sp_h.at[core, pl.ds(sub * N16, N16)])

    return kern(k0T, k1T, qcat, ei2, zeros)


def _sc_pass2(ev, srec, v0T, v1T, ei2, zeros):
    E = v0T.shape[2]
    N = srec.shape[0]
    C = 80
    NS = 4
    PW = E // _NW
    NCH = PW // C
    G = C // 16
    N16 = N // 16
    mesh = plsc.VectorSubcoreMesh(core_axis_name="c", subcore_axis_name="s")

    @functools.partial(
        pl.kernel,
        out_type=(jax.ShapeDtypeStruct((2, N, 8), jnp.float32),
                  jax.ShapeDtypeStruct((2, N, 24), jnp.float32)),
        mesh=mesh,
        compiler_params=pltpu.CompilerParams(use_tc_tiling_on_sc=False,
                                             needs_layout_passes=False),
        scratch_types=[
            pltpu.VMEM((C, 8), jnp.float32),
            pltpu.VMEM((1, _NH, C), jnp.float32),
            pltpu.VMEM((3, _NH, C), jnp.float32),
            pltpu.VMEM((C, 8), jnp.float32),
            pltpu.VMEM((C, 8), jnp.float32),
            pltpu.VMEM((C, 24), jnp.float32),
            pltpu.VMEM((1, _SB), jnp.int32),
            pltpu.VMEM((1, _SB), jnp.int32),
            pltpu.VMEM_SHARED((N, 8), jnp.float32),
            pltpu.VMEM_SHARED((N, 24), jnp.float32),
        ],
    )
    def kern(e_h, sr_h, v0_h, v1_h, ei_h, z_h, op0_h, op1_h,
             ecb, v0b, v1b, sgb, w0b, w1b, idxr, idxc, o0_sh, o1_sh):
        core = lax.axis_index("c")
        sub = lax.axis_index("s")
        wid = core * 16 + sub
        rw = wid * (PW // _SB)
        pltpu.sync_copy(z_h.at[pl.ds(sub * N16, N16), pl.ds(0, 8)],
                        o0_sh.at[pl.ds(sub * N16, N16)])
        pltpu.sync_copy(z_h.at[pl.ds(sub * N16, N16), pl.ds(0, 24)],
                        o1_sh.at[pl.ds(sub * N16, N16)])
        plsc.subcore_barrier()
        iota = lax.iota(jnp.int32, 16)

        def compute():
            @pl.loop(0, G)
            def _group(g):
                cv = iota + g * 16
                for h in range(_NH):
                    av = (plsc.load_gather(ecb, [cv, _splat(h)]) *
                          plsc.load_gather(sgb, [cv, _splat(h)]))
                    plsc.store_scatter(w0b, [cv, _splat(h)],
                                       av * v0b[0, h, pl.ds(g * 16, 16)])
                    for d in range(3):
                        plsc.store_scatter(
                            w1b, [cv, _splat(3 * h + d)],
                            av * v1b[d, h, pl.ds(g * 16, 16)])

        @pl.loop(0, NCH)
        def _chunk(i):
            eb = wid * PW + i * C
            rb = rw + i
            pltpu.sync_copy(e_h.at[pl.ds(eb, C)], ecb)
            pltpu.sync_copy(v0_h.at[:, :, pl.ds(eb, C)], v0b)
            pltpu.sync_copy(v1_h.at[:, :, pl.ds(eb, C)], v1b)
            pltpu.sync_copy(ei_h.at[pl.ds(rb, 1)], idxr)
            pltpu.sync_copy(ei_h.at[pl.ds(E // _SB + rb, 1)], idxc)
            pltpu.sync_copy(sr_h.at[idxr.at[0]], sgb)
            compute()
            pltpu.sync_copy(w0b, o0_sh.at[idxc.at[0]], add=True)
            pltpu.sync_copy(w1b, o1_sh.at[idxc.at[0]], add=True)

        plsc.subcore_barrier()
        pltpu.sync_copy(o0_sh.at[pl.ds(sub * N16, N16)],
                        op0_h.at[core, pl.ds(sub * N16, N16)])
        pltpu.sync_copy(o1_sh.at[pl.ds(sub * N16, N16)],
                        op1_h.at[core, pl.ds(sub * N16, N16)])

    return kern(ev, srec, v0T, v1T, ei2, zeros)


def _combine_s(sp):
    N = sp.shape[1]
    blk = 2000

    def body(x_ref, o_ref):
        o_ref[...] = 1.0 / (x_ref[0] + x_ref[1])

    return pl.pallas_call(
        body,
        grid=(N // blk,),
        in_specs=[pl.BlockSpec((2, blk, 8), lambda i: (0, i, 0))],
        out_specs=pl.BlockSpec((blk, 8), lambda i: (i, 0)),
        out_shape=jax.ShapeDtypeStruct((N, 8), jnp.float32),
    )(sp)


def _combine_out(op0, op1):
    N = op0.shape[1]
    blk = 2000

    def body(x0_ref, x1_ref, o0_ref, o1_ref):
        o0_ref[...] = x0_ref[0] + x0_ref[1]
        o1_ref[...] = x1_ref[0] + x1_ref[1]

    return pl.pallas_call(
        body,
        grid=(N // blk,),
        in_specs=[pl.BlockSpec((2, blk, 8), lambda i: (0, i, 0)),
                  pl.BlockSpec((2, blk, 24), lambda i: (0, i, 0))],
        out_specs=[pl.BlockSpec((blk, 8), lambda i: (i, 0)),
                   pl.BlockSpec((blk, 24), lambda i: (i, 0))],
        out_shape=(jax.ShapeDtypeStruct((N, 8), jnp.float32),
                   jax.ShapeDtypeStruct((N, 24), jnp.float32)),
    )(op0, op1)


def kernel(v_0, v_1, k_0, k_1, q_0, q_1, edge_index):
    E = v_0.shape[0]
    N = q_0.shape[0]
    k0T = jnp.transpose(k_0, (2, 1, 0))
    k1T = jnp.transpose(k_1, (2, 1, 0))
    v0T = jnp.transpose(v_0, (2, 1, 0))
    v1T = jnp.transpose(v_1, (2, 1, 0))
    qcat = jnp.concatenate([q_0, q_1], axis=-1).reshape(N, 32)
    ei2 = edge_index.reshape(2 * E // _SB, _SB)
    zeros = jnp.zeros((N, 32), jnp.float32)

    ev, sp = _sc_pass1(k0T, k1T, qcat, ei2, zeros)
    srec = _combine_s(sp)
    op0, op1 = _sc_pass2(ev, srec, v0T, v1T, ei2, zeros)
    o0, o1 = _combine_out(op0, op1)
    return (o0.reshape(N, 8, 1), o1.reshape(N, 8, 3))


# speedup vs baseline: 1.1186x; 1.1186x over previous
import functools
import math

import jax
import jax.numpy as jnp
from jax import lax
from jax.experimental import pallas as pl
from jax.experimental.pallas import tpu as pltpu
from jax.experimental.pallas import tpu_sc as plsc

_NH = 8
_SB = 80
_NW = 32
_INV = 1.0 / math.sqrt(32.0)


def _splat(v):
    return jnp.full((16,), v, jnp.int32)


def _sc_pass1(k0T, k1T, qcat, ei2, zeros):
    E = k0T.shape[2]
    N = qcat.shape[0]
    C = 400
    NSB = C // _SB
    PW = E // _NW
    NCH = PW // C
    G = C // 16
    N16 = N // 16
    mesh = plsc.VectorSubcoreMesh(core_axis_name="c", subcore_axis_name="s")

    @functools.partial(
        pl.kernel,
        out_type=(jax.ShapeDtypeStruct((E, _NH), jnp.float32),
                  jax.ShapeDtypeStruct((2, N, _NH), jnp.float32)),
        mesh=mesh,
        compiler_params=pltpu.CompilerParams(use_tc_tiling_on_sc=False,
                                             needs_layout_passes=False),
        scratch_types=[
            pltpu.VMEM((1, _NH, C), jnp.float32),
            pltpu.VMEM((3, _NH, C), jnp.float32),
            pltpu.VMEM((C, 32), jnp.float32),
            pltpu.VMEM((C, _NH), jnp.float32),
            pltpu.VMEM((NSB, _SB), jnp.int32),
            pltpu.VMEM((NSB, _SB), jnp.int32),
            pltpu.VMEM_SHARED((N, _NH), jnp.float32),
        ],
    )
    def kern(k0_h, k1_h, q_h, ei_h, z_h, e_h, sp_h,
             k0b, k1b, qgb, ecb, idxr, idxc, s_sh):
        core = lax.axis_index("c")
        sub = lax.axis_index("s")
        wid = core * 16 + sub
        rw = wid * (PW // _SB)
        pltpu.sync_copy(z_h.at[pl.ds(sub * N16, N16), pl.ds(0, _NH)],
                        s_sh.at[pl.ds(sub * N16, N16)])
        plsc.subcore_barrier()
        iota = lax.iota(jnp.int32, 16)

        def compute():
            @pl.loop(0, G)
            def _group(g):
                cv = iota + g * 16
                for h in range(_NH):
                    acc = (k0b[0, h, pl.ds(g * 16, 16)] *
                           plsc.load_gather(qgb, [cv, _splat(4 * h)]))
                    for d in range(3):
                        acc = acc + (
                            k1b[d, h, pl.ds(g * 16, 16)] *
                            plsc.load_gather(qgb,
                                             [cv, _splat(4 * h + 1 + d)]))
                    plsc.store_scatter(ecb, [cv, _splat(h)],
                                       jnp.exp(acc * _INV))

        @pl.loop(0, NCH)
        def _chunk(i):
            eb = wid * PW + i * C
            rb = rw + i * NSB
            pltpu.sync_copy(k0_h.at[:, :, pl.ds(eb, C)], k0b)
            pltpu.sync_copy(k1_h.at[:, :, pl.ds(eb, C)], k1b)
            pltpu.sync_copy(ei_h.at[pl.ds(rb, NSB)], idxr)
            pltpu.sync_copy(ei_h.at[pl.ds(E // _SB + rb, NSB)], idxc)
            for j in range(NSB):
                pltpu.sync_copy(q_h.at[idxc.at[j]],
                                qgb.at[pl.ds(j * _SB, _SB)])
            compute()
            pltpu.sync_copy(ecb, e_h.at[pl.ds(eb, C)])
            for j in range(NSB):
                pltpu.sync_copy(ecb.at[pl.ds(j * _SB, _SB)],
                                s_sh.at[idxr.at[j]], add=True)

        plsc.subcore_barrier()
        pltpu.sync_copy(s_sh.at[pl.ds(sub * N16, N16)],
                        sp_h.at[core, pl.ds(sub * N16, N16)])

    return kern(k0T, k1T, qcat, ei2, zeros)


def _sc_pass2(ev, srec, v0T, v1T, ei2, zeros):
    E = v0T.shape[2]
    N = srec.shape[0]
    C = 400
    NSB = C // _SB
    PW = E // _NW
    NCH = PW // C
    G = C // 16
    N16 = N // 16
    mesh = plsc.VectorSubcoreMesh(core_axis_name="c", subcore_axis_name="s")

    @functools.partial(
        pl.kernel,
        out_type=(jax.ShapeDtypeStruct((2, N, 8), jnp.float32),
                  jax.ShapeDtypeStruct((2, N, 24), jnp.float32)),
        mesh=mesh,
        compiler_params=pltpu.CompilerParams(use_tc_tiling_on_sc=False,
                                             needs_layout_passes=False),
        scratch_types=[
            pltpu.VMEM((C, 8), jnp.float32),
            pltpu.VMEM((1, _NH, C), jnp.float32),
            pltpu.VMEM((3, _NH, C), jnp.float32),
            pltpu.VMEM((C, 8), jnp.float32),
            pltpu.VMEM((C, 24), jnp.float32),
            pltpu.VMEM((NSB, _SB), jnp.int32),
            pltpu.VMEM((NSB, _SB), jnp.int32),
            pltpu.VMEM_SHARED((N, 8), jnp.float32),
            pltpu.VMEM_SHARED((N, 24), jnp.float32),
        ],
    )
    def kern(e_h, sr_h, v0_h, v1_h, ei_h, z_h, op0_h, op1_h,
             ecb, v0b, v1b, sgb, w1b, idxr, idxc, o0_sh, o1_sh):
        core = lax.axis_index("c")
        sub = lax.axis_index("s")
        wid = core * 16 + sub
        rw = wid * (PW // _SB)
        pltpu.sync_copy(z_h.at[pl.ds(sub * N16, N16), pl.ds(0, 8)],
                        o0_sh.at[pl.ds(sub * N16, N16)])
        pltpu.sync_copy(z_h.at[pl.ds(sub * N16, N16), pl.ds(0, 24)],
                        o1_sh.at[pl.ds(sub * N16, N16)])
        plsc.subcore_barrier()
        iota = lax.iota(jnp.int32, 16)

        def compute():
            @pl.loop(0, G)
            def _group(g):
                cv = iota + g * 16
                for h in range(_NH):
                    av = (plsc.load_gather(ecb, [cv, _splat(h)]) *
                          plsc.load_gather(sgb, [cv, _splat(h)]))
                    plsc.store_scatter(sgb, [cv, _splat(h)],
                                       av * v0b[0, h, pl.ds(g * 16, 16)])
                    for d in range(3):
                        plsc.store_scatter(
                            w1b, [cv, _splat(3 * h + d)],
                            av * v1b[d, h, pl.ds(g * 16, 16)])

        @pl.loop(0, NCH)
        def _chunk(i):
            eb = wid * PW + i * C
            rb = rw + i * NSB
            pltpu.sync_copy(e_h.at[pl.ds(eb, C)], ecb)
            pltpu.sync_copy(v0_h.at[:, :, pl.ds(eb, C)], v0b)
            pltpu.sync_copy(v1_h.at[:, :, pl.ds(eb, C)], v1b)
            pltpu.sync_copy(ei_h.at[pl.ds(rb, NSB)], idxr)
            pltpu.sync_copy(ei_h.at[pl.ds(E // _SB + rb, NSB)], idxc)
            for j in range(NSB):
                pltpu.sync_copy(sr_h.at[idxr.at[j]],
                                sgb.at[pl.ds(j * _SB, _SB)])
            compute()
            for j in range(NSB):
                pltpu.sync_copy(sgb.at[pl.ds(j * _SB, _SB)],
                                o0_sh.at[idxc.at[j]], add=True)
                pltpu.sync_copy(w1b.at[pl.ds(j * _SB, _SB)],
                                o1_sh.at[idxc.at[j]], add=True)

        plsc.subcore_barrier()
        pltpu.sync_copy(o0_sh.at[pl.ds(sub * N16, N16)],
                        op0_h.at[core, pl.ds(sub * N16, N16)])
        pltpu.sync_copy(o1_sh.at[pl.ds(sub * N16, N16)],
                        op1_h.at[core, pl.ds(sub * N16, N16)])

    return kern(ev, srec, v0T, v1T, ei2, zeros)


def _combine_s(sp):
    N = sp.shape[1]
    blk = 2000

    def body(x_ref, o_ref):
        o_ref[...] = 1.0 / (x_ref[0] + x_ref[1])

    return pl.pallas_call(
        body,
        grid=(N // blk,),
        in_specs=[pl.BlockSpec((2, blk, 8), lambda i: (0, i, 0))],
        out_specs=pl.BlockSpec((blk, 8), lambda i: (i, 0)),
        out_shape=jax.ShapeDtypeStruct((N, 8), jnp.float32),
    )(sp)


def _combine_out(op0, op1):
    N = op0.shape[1]
    blk = 2000

    def body(x0_ref, x1_ref, o0_ref, o1_ref):
        o0_ref[...] = x0_ref[0] + x0_ref[1]
        o1_ref[...] = x1_ref[0] + x1_ref[1]

    return pl.pallas_call(
        body,
        grid=(N // blk,),
        in_specs=[pl.BlockSpec((2, blk, 8), lambda i: (0, i, 0)),
                  pl.BlockSpec((2, blk, 24), lambda i: (0, i, 0))],
        out_specs=[pl.BlockSpec((blk, 8), lambda i: (i, 0)),
                   pl.BlockSpec((blk, 24), lambda i: (i, 0))],
        out_shape=(jax.ShapeDtypeStruct((N, 8), jnp.float32),
                   jax.ShapeDtypeStruct((N, 24), jnp.float32)),
    )(op0, op1)


def kernel(v_0, v_1, k_0, k_1, q_0, q_1, edge_index):
    E = v_0.shape[0]
    N = q_0.shape[0]
    k0T = jnp.transpose(k_0, (2, 1, 0))
    k1T = jnp.transpose(k_1, (2, 1, 0))
    v0T = jnp.transpose(v_0, (2, 1, 0))
    v1T = jnp.transpose(v_1, (2, 1, 0))
    qcat = jnp.concatenate([q_0, q_1], axis=-1).reshape(N, 32)
    ei2 = edge_index.reshape(2 * E // _SB, _SB)
    zeros = jnp.zeros((N, 32), jnp.float32)

    ev, sp = _sc_pass1(k0T, k1T, qcat, ei2, zeros)
    srec = _combine_s(sp)
    op0, op1 = _sc_pass2(ev, srec, v0T, v1T, ei2, zeros)
    o0, o1 = _combine_out(op0, op1)
    return (o0.reshape(N, 8, 1), o1.reshape(N, 8, 3))


# speedup vs baseline: 1.6338x; 1.4605x over previous
import functools
import math

import jax
import jax.numpy as jnp
from jax import lax
from jax.experimental import pallas as pl
from jax.experimental.pallas import tpu as pltpu
from jax.experimental.pallas import tpu_sc as plsc

_NH = 8
_FK = 4
_SB = 100
_C = 400
_NSB = _C // _SB
_NW = 32
_INV = 1.0 / math.sqrt(32.0)


def _splat(v):
    return jnp.full((16,), v, jnp.int32)


def _sc_pass1(k0, k1, qcat, ei3, zeros):
    E = k0.shape[0]
    N = qcat.shape[0]
    PW = E // _NW
    NCH = PW // _C
    G = _C // 16
    N16 = N // 16
    mesh = plsc.VectorSubcoreMesh(core_axis_name="c", subcore_axis_name="s")

    @functools.partial(
        pl.kernel,
        out_type=(jax.ShapeDtypeStruct((E, _NH), jnp.float32),
                  jax.ShapeDtypeStruct((2, N, _NH), jnp.float32)),
        mesh=mesh,
        compiler_params=pltpu.CompilerParams(use_tc_tiling_on_sc=False,
                                             needs_layout_passes=False),
        scratch_types=[
            pltpu.VMEM((_C, 8), jnp.float32),
            pltpu.VMEM((_C, 24), jnp.float32),
            pltpu.VMEM((_C, 32), jnp.float32),
            pltpu.VMEM((_C, 8), jnp.float32),
            pltpu.VMEM((2, _NSB, _SB), jnp.int32),
            pltpu.VMEM_SHARED((N, _NH), jnp.float32),
        ],
    )
    def kern(k0_h, k1_h, q_h, ei_h, z_h, e_h, sp_h,
             k0b, k1b, qgb, ecb, idxb, s_sh):
        core = lax.axis_index("c")
        sub = lax.axis_index("s")
        wid = core * 16 + sub
        pltpu.sync_copy(z_h.at[pl.ds(sub * N16, N16), pl.ds(0, _NH)],
                        s_sh.at[pl.ds(sub * N16, N16)])
        plsc.subcore_barrier()
        iota = lax.iota(jnp.int32, 16)

        @pl.loop(0, NCH)
        def _chunk(i):
            eb = wid * PW + i * _C
            rb = wid * (PW // _SB) + i * _NSB
            pltpu.sync_copy(k0_h.at[pl.ds(eb, _C)], k0b)
            pltpu.sync_copy(k1_h.at[pl.ds(eb, _C)], k1b)
            pltpu.sync_copy(ei_h.at[:, pl.ds(rb, _NSB)], idxb)
            for j in range(_NSB):
                pltpu.sync_copy(q_h.at[idxb.at[1, j]],
                                qgb.at[pl.ds(j * _SB, _SB)])

            @pl.loop(0, G)
            def _group(g):
                cv = iota + g * 16
                for h in range(_NH):
                    acc = (plsc.load_gather(k0b, [cv, _splat(h)]) *
                           plsc.load_gather(qgb, [cv, _splat(4 * h)]))
                    for d in range(3):
                        acc = acc + (
                            plsc.load_gather(k1b, [cv, _splat(3 * h + d)]) *
                            plsc.load_gather(qgb, [cv, _splat(4 * h + 1 + d)]))
                    plsc.store_scatter(ecb, [cv, _splat(h)],
                                       jnp.exp(acc * _INV))

            pltpu.sync_copy(ecb, e_h.at[pl.ds(eb, _C)])
            for j in range(_NSB):
                pltpu.sync_copy(ecb.at[pl.ds(j * _SB, _SB)],
                                s_sh.at[idxb.at[0, j]], add=True)

        plsc.subcore_barrier()
        pltpu.sync_copy(s_sh.at[pl.ds(sub * N16, N16)],
                        sp_h.at[core, pl.ds(sub * N16, N16)])

    return kern(k0, k1, qcat, ei3, zeros)


def _sc_pass2(ev, sv, v0, v1, ei3, zeros):
    E = v0.shape[0]
    N = sv.shape[0]
    PW = E // _NW
    NCH = PW // _C
    G = _C // 16
    N16 = N // 16
    mesh = plsc.VectorSubcoreMesh(core_axis_name="c", subcore_axis_name="s")

    @functools.partial(
        pl.kernel,
        out_type=(jax.ShapeDtypeStruct((2, N, 8), jnp.float32),
                  jax.ShapeDtypeStruct((2, N, 24), jnp.float32)),
        mesh=mesh,
        compiler_params=pltpu.CompilerParams(use_tc_tiling_on_sc=False,
                                             needs_layout_passes=False),
        scratch_types=[
            pltpu.VMEM((_C, 8), jnp.float32),
            pltpu.VMEM((_C, 8), jnp.float32),
            pltpu.VMEM((_C, 24), jnp.float32),
            pltpu.VMEM((_C, 8), jnp.float32),
            pltpu.VMEM((2, _NSB, _SB), jnp.int32),
            pltpu.VMEM_SHARED((N, 8), jnp.float32),
            pltpu.VMEM_SHARED((N, 24), jnp.float32),
        ],
    )
    def kern(e_h, s_h, v0_h, v1_h, ei_h, z_h, op0_h, op1_h,
             ecb, v0b, v1b, sgb, idxb, o0_sh, o1_sh):
        core = lax.axis_index("c")
        sub = lax.axis_index("s")
        wid = core * 16 + sub
        pltpu.sync_copy(z_h.at[pl.ds(sub * N16, N16), pl.ds(0, 8)],
                        o0_sh.at[pl.ds(sub * N16, N16)])
        pltpu.sync_copy(z_h.at[pl.ds(sub * N16, N16), pl.ds(0, 24)],
                        o1_sh.at[pl.ds(sub * N16, N16)])
        plsc.subcore_barrier()
        iota = lax.iota(jnp.int32, 16)

        @pl.loop(0, NCH)
        def _chunk(i):
            eb = wid * PW + i * _C
            rb = wid * (PW // _SB) + i * _NSB
            pltpu.sync_copy(e_h.at[pl.ds(eb, _C)], ecb)
            pltpu.sync_copy(v0_h.at[pl.ds(eb, _C)], v0b)
            pltpu.sync_copy(v1_h.at[pl.ds(eb, _C)], v1b)
            pltpu.sync_copy(ei_h.at[:, pl.ds(rb, _NSB)], idxb)
            for j in range(_NSB):
                pltpu.sync_copy(s_h.at[idxb.at[0, j]],
                                sgb.at[pl.ds(j * _SB, _SB)])

            @pl.loop(0, G)
            def _group(g):
                cv = iota + g * 16
                for h in range(_NH):
                    av = (plsc.load_gather(ecb, [cv, _splat(h)]) /
                          plsc.load_gather(sgb, [cv, _splat(h)]))
                    plsc.store_scatter(
                        v0b, [cv, _splat(h)],
                        av * plsc.load_gather(v0b, [cv, _splat(h)]))
                    for d in range(3):
                        plsc.store_scatter(
                            v1b, [cv, _splat(3 * h + d)],
                            av * plsc.load_gather(v1b, [cv, _splat(3 * h + d)]))

            for j in range(_NSB):
                pltpu.sync_copy(v0b.at[pl.ds(j * _SB, _SB)],
                                o0_sh.at[idxb.at[1, j]], add=True)
                pltpu.sync_copy(v1b.at[pl.ds(j * _SB, _SB)],
                                o1_sh.at[idxb.at[1, j]], add=True)

        plsc.subcore_barrier()
        pltpu.sync_copy(o0_sh.at[pl.ds(sub * N16, N16)],
                        op0_h.at[core, pl.ds(sub * N16, N16)])
        pltpu.sync_copy(o1_sh.at[pl.ds(sub * N16, N16)],
                        op1_h.at[core, pl.ds(sub * N16, N16)])

    return kern(ev, sv, v0, v1, ei3, zeros)


def _combine_s(sp):
    N = sp.shape[1]
    blk = 2000

    def body(x_ref, o_ref):
        o_ref[...] = x_ref[0] + x_ref[1]

    return pl.pallas_call(
        body,
        grid=(N // blk,),
        in_specs=[pl.BlockSpec((2, blk, 8), lambda i: (0, i, 0))],
        out_specs=pl.BlockSpec((blk, 8), lambda i: (i, 0)),
        out_shape=jax.ShapeDtypeStruct((N, 8), jnp.float32),
    )(sp)


def _combine_out(op0, op1):
    N = op0.shape[1]
    blk = 2000

    def body(x0_ref, x1_ref, o0_ref, o1_ref):
        o0_ref[...] = x0_ref[0] + x0_ref[1]
        o1_ref[...] = x1_ref[0] + x1_ref[1]

    return pl.pallas_call(
        body,
        grid=(N // blk,),
        in_specs=[pl.BlockSpec((2, blk, 8), lambda i: (0, i, 0)),
                  pl.BlockSpec((2, blk, 24), lambda i: (0, i, 0))],
        out_specs=[pl.BlockSpec((blk, 8), lambda i: (i, 0)),
                   pl.BlockSpec((blk, 24), lambda i: (i, 0))],
        out_shape=(jax.ShapeDtypeStruct((N, 8), jnp.float32),
                   jax.ShapeDtypeStruct((N, 24), jnp.float32)),
    )(op0, op1)


def kernel(v_0, v_1, k_0, k_1, q_0, q_1, edge_index):
    E = v_0.shape[0]
    N = q_0.shape[0]
    k0 = k_0.reshape(E, 8)
    k1 = k_1.reshape(E, 24)
    v0 = v_0.reshape(E, 8)
    v1 = v_1.reshape(E, 24)
    qcat = jnp.concatenate([q_0, q_1], axis=-1).reshape(N, 32)
    ei3 = edge_index.reshape(2, E // _SB, _SB)
    zeros = jnp.zeros((N, 32), jnp.float32)

    ev, sp = _sc_pass1(k0, k1, qcat, ei3, zeros)
    s = _combine_s(sp)
    op0, op1 = _sc_pass2(ev, s, v0, v1, ei3, zeros)
    o0, o1 = _combine_out(op0, op1)
    return (o0.reshape(N, 8, 1), o1.reshape(N, 8, 3))
